# baseline (speedup 1.0000x reference)
import functools

import jax
import jax.numpy as jnp
from jax.experimental import pallas as pl
from jax.experimental.pallas import tpu as pltpu

N = 16384
H = 2048
D = 64

NBUF = 3


def _fused_kernel(nh, nm, th, tm, tar_hbm, embs_hbm, src_hbm, o_hbm,
                  embs_v, ebf, acc, abf, tbuf, sbuf, obuf,
                  esem, tsem, ssem, osem):
    def tar_copy(j, slot):
        return pltpu.make_async_copy(
            tar_hbm.at[pl.ds(j * th, th), :], tbuf.at[slot], tsem.at[slot])

    def src_copy(m, slot):
        return pltpu.make_async_copy(
            src_hbm.at[pl.ds(m * tm, tm), :], sbuf.at[slot], ssem.at[slot])

    def out_copy(m, slot):
        return pltpu.make_async_copy(
            obuf.at[slot], o_hbm.at[pl.ds(m * tm, tm), :], osem.at[slot])

    ecopy = pltpu.make_async_copy(embs_hbm, embs_v, esem)
    ecopy.start()
    tar_copy(0, 0).start()
    tar_copy(1, 1).start()
    ecopy.wait()
    ebf[...] = embs_v[...].astype(jnp.bfloat16)

    def phase1(j, carry):
        slot = jax.lax.rem(j, NBUF)
        tar_copy(j, slot).wait()

        @pl.when(j + 2 < nh)
        def _next():
            tar_copy(j + 2, jax.lax.rem(j + 2, NBUF)).start()

        @pl.when(j == nh - 2)
        def _warm0():
            src_copy(0, 0).start()

        @pl.when(j == nh - 1)
        def _warm1():
            src_copy(1, 1).start()

        acc[pl.ds(j * th, th), :] = jnp.dot(
            tbuf[slot].astype(jnp.bfloat16), ebf[...],
            preferred_element_type=jnp.float32)
        return carry

    jax.lax.fori_loop(0, nh, phase1, 0)
    abf[...] = acc[...].astype(jnp.bfloat16)

    def phase2(m, carry):
        slot = jax.lax.rem(m, NBUF)
        oslot = jax.lax.rem(m, 2)
        src_copy(m, slot).wait()

        @pl.when(m + 2 < nm)
        def _next():
            src_copy(m + 2, jax.lax.rem(m + 2, NBUF)).start()

        @pl.when(m >= 2)
        def _drain():
            out_copy(m - 2, oslot).wait()

        obuf[oslot] = jnp.maximum(
            jnp.dot(sbuf[slot].astype(jnp.bfloat16), abf[...],
                    preferred_element_type=jnp.float32),
            0.0)
        out_copy(m, oslot).start()
        return carry

    jax.lax.fori_loop(0, nm, phase2, 0)
    out_copy(nm - 2, jax.lax.rem(nm - 2, 2)).wait()
    out_copy(nm - 1, jax.lax.rem(nm - 1, 2)).wait()


@functools.partial(jax.jit, static_argnames=("th", "tm"))
def _run(pois_embs, HG_poi_src, HG_poi_tar, th=128, tm=512):
    nh = H // th
    nm = N // tm
    any_spec = pl.BlockSpec(memory_space=pltpu.MemorySpace.HBM)
    return pl.pallas_call(
        functools.partial(_fused_kernel, nh, nm, th, tm),
        in_specs=[any_spec, any_spec, any_spec],
        out_specs=any_spec,
        out_shape=jax.ShapeDtypeStruct((N, D), jnp.float32),
        scratch_shapes=[
            pltpu.VMEM((N, D), jnp.float32),
            pltpu.VMEM((N, D), jnp.bfloat16),
            pltpu.VMEM((H, D), jnp.float32),
            pltpu.VMEM((H, D), jnp.bfloat16),
            pltpu.VMEM((NBUF, th, N), jnp.float32),
            pltpu.VMEM((NBUF, tm, H), jnp.float32),
            pltpu.VMEM((2, tm, D), jnp.float32),
            pltpu.SemaphoreType.DMA,
            pltpu.SemaphoreType.DMA((NBUF,)),
            pltpu.SemaphoreType.DMA((NBUF,)),
            pltpu.SemaphoreType.DMA((2,)),
        ],
        compiler_params=pltpu.CompilerParams(
            vmem_limit_bytes=63 * 1024 * 1024),
    )(HG_poi_tar, pois_embs, HG_poi_src)


def kernel(pois_embs, HG_poi_src, HG_poi_tar):
    return _run(pois_embs, HG_poi_src, HG_poi_tar)


# speedup vs baseline: 1.0304x; 1.0304x over previous
import functools

import jax
import jax.numpy as jnp
from jax.experimental import pallas as pl
from jax.experimental.pallas import tpu as pltpu

N = 16384
H = 2048
D = 64

NBUF = 3


def _fused_kernel(nh, nm, th, tm, tar_hbm, embs_hbm, src_hbm, o_hbm,
                  ebf, acc, abf, tbuf, sbuf, obuf,
                  esem, tsem, ssem, osem):
    def tar_copy(j, slot):
        return pltpu.make_async_copy(
            tar_hbm.at[pl.ds(j * th, th), :], tbuf.at[slot], tsem.at[slot])

    def src_copy(m, slot):
        return pltpu.make_async_copy(
            src_hbm.at[pl.ds(m * tm, tm), :], sbuf.at[slot], ssem.at[slot])

    def out_copy(m, slot):
        return pltpu.make_async_copy(
            obuf.at[slot], o_hbm.at[pl.ds(m * tm, tm), :], osem.at[slot])

    ecopy = pltpu.make_async_copy(embs_hbm, ebf, esem)
    ecopy.start()
    tar_copy(0, 0).start()
    tar_copy(1, 1).start()
    ecopy.wait()

    def phase1(j, carry):
        slot = jax.lax.rem(j, NBUF)
        tar_copy(j, slot).wait()

        @pl.when(j + 2 < nh)
        def _next():
            tar_copy(j + 2, jax.lax.rem(j + 2, NBUF)).start()

        @pl.when(j == nh - 2)
        def _warm0():
            src_copy(0, 0).start()

        @pl.when(j == nh - 1)
        def _warm1():
            src_copy(1, 1).start()

        acc[pl.ds(j * th, th), :] = jnp.dot(
            tbuf[slot].astype(jnp.bfloat16), ebf[...],
            preferred_element_type=jnp.float32)
        return carry

    jax.lax.fori_loop(0, nh, phase1, 0)
    abf[...] = acc[...].astype(jnp.bfloat16)

    def phase2(m, carry):
        slot = jax.lax.rem(m, NBUF)
        oslot = jax.lax.rem(m, 2)
        src_copy(m, slot).wait()

        @pl.when(m + 2 < nm)
        def _next():
            src_copy(m + 2, jax.lax.rem(m + 2, NBUF)).start()

        @pl.when(m >= 2)
        def _drain():
            out_copy(m - 2, oslot).wait()

        obuf[oslot] = jnp.maximum(
            jnp.dot(sbuf[slot].astype(jnp.bfloat16), abf[...],
                    preferred_element_type=jnp.float32),
            0.0)
        out_copy(m, oslot).start()
        return carry

    jax.lax.fori_loop(0, nm, phase2, 0)
    out_copy(nm - 2, jax.lax.rem(nm - 2, 2)).wait()
    out_copy(nm - 1, jax.lax.rem(nm - 1, 2)).wait()


@functools.partial(jax.jit, static_argnames=("th", "tm"))
def _run(embs_bf, HG_poi_src, HG_poi_tar, th=128, tm=1024):
    nh = H // th
    nm = N // tm
    any_spec = pl.BlockSpec(memory_space=pltpu.MemorySpace.HBM)
    return pl.pallas_call(
        functools.partial(_fused_kernel, nh, nm, th, tm),
        in_specs=[any_spec, any_spec, any_spec],
        out_specs=any_spec,
        out_shape=jax.ShapeDtypeStruct((N, D), jnp.float32),
        scratch_shapes=[
            pltpu.VMEM((N, D), jnp.bfloat16),
            pltpu.VMEM((H, D), jnp.float32),
            pltpu.VMEM((H, D), jnp.bfloat16),
            pltpu.VMEM((NBUF, th, N), jnp.float32),
            pltpu.VMEM((NBUF, tm, H), jnp.float32),
            pltpu.VMEM((2, tm, D), jnp.float32),
            pltpu.SemaphoreType.DMA,
            pltpu.SemaphoreType.DMA((NBUF,)),
            pltpu.SemaphoreType.DMA((NBUF,)),
            pltpu.SemaphoreType.DMA((2,)),
        ],
        compiler_params=pltpu.CompilerParams(
            vmem_limit_bytes=63 * 1024 * 1024),
    )(HG_poi_tar, embs_bf, HG_poi_src)


def kernel(pois_embs, HG_poi_src, HG_poi_tar):
    return _run(pois_embs.astype(jnp.bfloat16), HG_poi_src, HG_poi_tar)
